# baseline (speedup 1.0000x reference)
import functools

import jax
import jax.numpy as jnp
from jax import lax
from jax.experimental import pallas as pl
from jax.experimental.pallas import tpu as pltpu
from jax.experimental.pallas import tpu_sc as plsc

_N = 4_000_000
_GN = 256
_NCELL = _GN * _GN * _GN
_SLAB = 1_835_008
_RUNT = 1 << 20
_RUNT_BASE = 8 * _SLAB
_N_PAD = 1 << 22
_NTILE = 16
_TILE_SHARE = _N_PAD // _NTILE
_W = 2048
_NWIN = _TILE_SHARE // _W
_UNROLL = 8
_CH = 128
_CB = _W + 256


def _cell_idx_body(x_ref, y_ref, z_ref, m_ref, idx_ref, w_ref):
    gmin = jnp.float32(-10.0)
    dx = (jnp.float32(10.0) - gmin) / jnp.float32(_GN - 1)

    def cell(p):
        fi = (p - gmin) / dx + jnp.float32(0.5)
        return jnp.clip(fi, -4.0, 300.0).astype(jnp.int32)

    ix = cell(x_ref[...])
    iy = cell(y_ref[...])
    iz = cell(z_ref[...])
    ok = ((ix >= 0) & (ix < _GN) & (iy >= 0) & (iy < _GN)
          & (iz >= 0) & (iz < _GN))
    flat = (ix * _GN + iy) * _GN + iz
    shp = idx_ref.shape
    dummy = (lax.broadcasted_iota(jnp.int32, shp, 0) * shp[1]
             + lax.broadcasted_iota(jnp.int32, shp, 1))
    idx_ref[...] = jnp.where(ok, flat, dummy & (_NCELL - 1))
    w_ref[...] = jnp.where(ok, m_ref[...], jnp.float32(0.0))


def _cell_idx(x, y, z, m):
    rows, cols = 4096, 1024
    blk = 512
    npad = _N_PAD - _N
    x2, y2, z2 = (jnp.pad(a, (0, npad), constant_values=1e9)
                  .reshape(rows, cols) for a in (x, y, z))
    m2 = jnp.pad(m, (0, npad)).reshape(rows, cols)
    spec = pl.BlockSpec((blk, cols), lambda i: (i, 0))
    idx2, w2 = pl.pallas_call(
        _cell_idx_body,
        grid=(rows // blk,),
        in_specs=[spec] * 4,
        out_specs=[spec] * 2,
        out_shape=[
            jax.ShapeDtypeStruct((rows, cols), jnp.int32),
            jax.ShapeDtypeStruct((rows, cols), jnp.float32),
        ],
    )(x2, y2, z2, m2)
    return idx2.reshape(-1), w2.reshape(-1)


@functools.partial(
    pl.kernel,
    mesh=plsc.VectorSubcoreMesh(core_axis_name="c", subcore_axis_name="s"),
    out_type=jax.ShapeDtypeStruct((_NCELL,), jnp.float32),
    scratch_types=[
        pltpu.VMEM((_W,), jnp.int32),
        pltpu.VMEM((_W,), jnp.float32),
        pltpu.VMEM((_CB,), jnp.int32),
        pltpu.VMEM((_CB,), jnp.float32),
        pltpu.VMEM_SHARED((_SLAB,), jnp.float32),
    ],
    compiler_params=pltpu.CompilerParams(needs_layout_passes=False),
)
def _scatter_kernel(idx_hbm, w_hbm, out_hbm, idxw, ww, ci, cw, acc):
    c = lax.axis_index("c")
    s = lax.axis_index("s")
    pbase = s * _TILE_SHARE
    iota = lax.iota(jnp.int32, 16)
    zv = jnp.zeros((16,), jnp.float32)
    ones = jnp.ones((16,), jnp.bool_)

    def slab_pass(slab_base, size):
        chunk = size // _NTILE

        def zfill(i, _):
            ww[pl.ds(i * 16, 16)] = zv
            return 0

        lax.fori_loop(0, _W // 16, zfill, 0)
        for k in range(chunk // _W):
            pltpu.sync_copy(ww, acc.at[pl.ds(s * chunk + k * _W, _W)])
        plsc.subcore_barrier()

        def vbody(i, cur):
            for u in range(_UNROLL):
                b = i * (16 * _UNROLL) + u * 16
                lv = idxw[pl.ds(b, 16)]
                loc = lv - slab_base
                m = (loc >= 0) & (loc < size)
                plsc.store_compressed(ci.at[pl.ds(cur, 16)], loc, mask=m)
                wv = ww[pl.ds(b, 16)]
                plsc.store_compressed(cw.at[pl.ds(cur, 16)], wv, mask=m)
                cur = cur + plsc.all_reduce_population_count(m)[0]
            return cur

        def flush(t, _):
            pltpu.sync_copy(cw.at[pl.ds(t * _CH, _CH)],
                            acc.at[ci.at[pl.ds(t * _CH, _CH)]], add=True)
            return 0

        def wbody(win, cur):
            off = pbase + win * _W
            pltpu.sync_copy(idx_hbm.at[pl.ds(off, _W)], idxw)
            pltpu.sync_copy(w_hbm.at[pl.ds(off, _W)], ww)
            cur = lax.fori_loop(0, _W // (16 * _UNROLL), vbody, cur)
            nfl = cur // _CH
            lax.fori_loop(0, nfl, flush, 0)
            base = nfl * _CH
            for t in range(_CH // 16):
                ci[pl.ds(t * 16, 16)] = ci[pl.ds(base + t * 16, 16)]
                cw[pl.ds(t * 16, 16)] = cw[pl.ds(base + t * 16, 16)]
            return cur - base

        cur = lax.fori_loop(0, _NWIN, wbody, jnp.int32(0))
        for t in range(_CH // 16):
            plsc.store_compressed(ci.at[pl.ds(cur + t * 16, 16)],
                                  iota + t * 16, mask=ones)
            plsc.store_compressed(cw.at[pl.ds(cur + t * 16, 16)], zv,
                                  mask=ones)
        flush(0, 0)
        plsc.subcore_barrier()

        pltpu.sync_copy(acc.at[pl.ds(s * chunk, chunk)],
                        out_hbm.at[pl.ds(slab_base + s * chunk, chunk)])
        plsc.subcore_barrier()

    for j in range(4):
        slab_pass((c * 4 + j) * _SLAB, _SLAB)
    slab_pass(_RUNT_BASE + c * _RUNT, _RUNT)


def kernel(positions, masses):
    pt = positions.T
    idx, w = _cell_idx(pt[0], pt[1], pt[2], masses)
    grid = _scatter_kernel(idx, w)
    return grid.reshape(_GN, _GN, _GN)


# speedup vs baseline: 1.1460x; 1.1460x over previous
import functools

import jax
import jax.numpy as jnp
from jax import lax
from jax.experimental import pallas as pl
from jax.experimental.pallas import tpu as pltpu
from jax.experimental.pallas import tpu_sc as plsc

_N = 4_000_000
_GN = 256
_NCELL = _GN * _GN * _GN
_SLAB = 1_835_008
_RUNT = 1 << 20
_RUNT_BASE = 8 * _SLAB
_N_PAD = 1 << 22
_NTILE = 16
_TILE_SHARE = _N_PAD // _NTILE
_W = 4096
_NWIN = _TILE_SHARE // _W
_UNROLL = 8


def _cell_idx_body(x_ref, y_ref, z_ref, m_ref, idx_ref, w_ref):
    gmin = jnp.float32(-10.0)
    dx = (jnp.float32(10.0) - gmin) / jnp.float32(_GN - 1)

    def cell(p):
        fi = (p - gmin) / dx + jnp.float32(0.5)
        return jnp.clip(fi, -4.0, 300.0).astype(jnp.int32)

    ix = cell(x_ref[...])
    iy = cell(y_ref[...])
    iz = cell(z_ref[...])
    ok = ((ix >= 0) & (ix < _GN) & (iy >= 0) & (iy < _GN)
          & (iz >= 0) & (iz < _GN))
    flat = (ix * _GN + iy) * _GN + iz
    shp = idx_ref.shape
    dummy = (lax.broadcasted_iota(jnp.int32, shp, 0) * shp[1]
             + lax.broadcasted_iota(jnp.int32, shp, 1))
    idx_ref[...] = jnp.where(ok, flat, dummy & (_NCELL - 1))
    w_ref[...] = jnp.where(ok, m_ref[...], jnp.float32(0.0))


def _cell_idx(x, y, z, m):
    rows, cols = 4096, 1024
    blk = 512
    npad = _N_PAD - _N
    x2, y2, z2 = (jnp.pad(a, (0, npad), constant_values=1e9)
                  .reshape(rows, cols) for a in (x, y, z))
    m2 = jnp.pad(m, (0, npad)).reshape(rows, cols)
    spec = pl.BlockSpec((blk, cols), lambda i: (i, 0))
    idx2, w2 = pl.pallas_call(
        _cell_idx_body,
        grid=(rows // blk,),
        in_specs=[spec] * 4,
        out_specs=[spec] * 2,
        out_shape=[
            jax.ShapeDtypeStruct((rows, cols), jnp.int32),
            jax.ShapeDtypeStruct((rows, cols), jnp.float32),
        ],
    )(x2, y2, z2, m2)
    return idx2.reshape(-1), w2.reshape(-1)


@functools.partial(
    pl.kernel,
    mesh=plsc.VectorSubcoreMesh(core_axis_name="c", subcore_axis_name="s"),
    out_type=jax.ShapeDtypeStruct((_NCELL,), jnp.float32),
    scratch_types=[
        pltpu.VMEM((_W,), jnp.int32),
        pltpu.VMEM((_W,), jnp.float32),
        pltpu.VMEM_SHARED((_SLAB,), jnp.float32),
    ],
    compiler_params=pltpu.CompilerParams(needs_layout_passes=False),
)
def _scatter_kernel(idx_hbm, w_hbm, out_hbm, idxw, ww, acc):
    c = lax.axis_index("c")
    s = lax.axis_index("s")
    pbase = s * _TILE_SHARE
    iota = lax.iota(jnp.int32, 16)
    zv = jnp.zeros((16,), jnp.float32)

    def slab_pass(slab_base, size):
        chunk = size // _NTILE

        def zfill(i, _):
            ww[pl.ds(i * 16, 16)] = zv
            return 0

        lax.fori_loop(0, _W // 16, zfill, 0)
        for k in range(chunk // _W):
            pltpu.sync_copy(ww, acc.at[pl.ds(s * chunk + k * _W, _W)])
        plsc.subcore_barrier()

        def vbody(i, _):
            for u in range(_UNROLL):
                b = i * (16 * _UNROLL) + u * 16
                lv = idxw[pl.ds(b, 16)]
                loc = lv - slab_base
                m = (loc >= 0) & (loc < size)
                idxw[pl.ds(b, 16)] = jnp.where(m, loc, iota + b)
                wv = ww[pl.ds(b, 16)]
                ww[pl.ds(b, 16)] = jnp.where(m, wv, jnp.float32(0.0))
            return 0

        def wbody(win, _):
            off = pbase + win * _W
            pltpu.sync_copy(idx_hbm.at[pl.ds(off, _W)], idxw)
            pltpu.sync_copy(w_hbm.at[pl.ds(off, _W)], ww)
            lax.fori_loop(0, _W // (16 * _UNROLL), vbody, 0)
            pltpu.sync_copy(ww, acc.at[idxw], add=True)
            return 0

        lax.fori_loop(0, _NWIN, wbody, 0)
        plsc.subcore_barrier()

        pltpu.sync_copy(acc.at[pl.ds(s * chunk, chunk)],
                        out_hbm.at[pl.ds(slab_base + s * chunk, chunk)])
        plsc.subcore_barrier()

    for j in range(4):
        slab_pass((c * 4 + j) * _SLAB, _SLAB)
    slab_pass(_RUNT_BASE + c * _RUNT, _RUNT)


def kernel(positions, masses):
    pt = positions.T
    idx, w = _cell_idx(pt[0], pt[1], pt[2], masses)
    grid = _scatter_kernel(idx, w)
    return grid.reshape(_GN, _GN, _GN)
